# Initial kernel scaffold; baseline (speedup 1.0000x reference)
import jax
import jax.numpy as jnp
from jax.experimental import pallas as pl


def kernel(output, target):
    raise NotImplementedError("write your pallas kernel here")


# speedup vs baseline: 35.1662x; 35.1662x over previous
import jax
import jax.numpy as jnp
from jax import lax
from jax.experimental import pallas as pl
from jax.experimental.pallas import tpu as pltpu
from jax.experimental.pallas import tpu_sc as plsc

NC = 2
NS = 16
L = 16
ROWS = 16
COLS = 1048576
NBINS = 100
PAD = 128
CH = 32768
NCH = COLS // CH


def _sc_body(out_hbm, tgt_hbm, ph_hbm, tp_hbm,
             buf, hist, red, mm, mmall, smin, smax):
    cid = lax.axis_index("c")
    sid = lax.axis_index("s")

    def stage(j):
        off = pl.multiple_of(j * CH, 8)

        @pl.when(cid == 0)
        def _():
            pltpu.sync_copy(out_hbm.at[sid, pl.ds(off, CH)], buf)

        @pl.when(cid == 1)
        def _():
            pltpu.sync_copy(tgt_hbm.at[sid, pl.ds(off, CH)], buf)

    def mm_chunk(j, carry):
        stage(j)

        def vbody(i, c):
            x = buf[pl.ds(pl.multiple_of(i * L, 8), L)]
            return jnp.minimum(c[0], x), jnp.maximum(c[1], x)

        return lax.fori_loop(0, CH // L, vbody, carry, unroll=8)

    init = (jnp.full((L,), jnp.inf, jnp.float32),
            jnp.full((L,), -jnp.inf, jnp.float32))
    vmn, vmx = lax.fori_loop(0, NCH, mm_chunk, init)

    gdn = lax.GatherDimensionNumbers(offset_dims=(), collapsed_slice_dims=(0,),
                                     start_index_map=(0,))

    def _xlane(v, op):
        for p in (8, 4, 2, 1):
            perm = (lax.broadcasted_iota(jnp.int32, (L,), 0) ^ p).reshape(L, 1)
            v = op(v, lax.gather(v, perm, gdn, slice_sizes=(1,),
                                 mode=lax.GatherScatterMode.PROMISE_IN_BOUNDS))
        return v

    lo_own = _xlane(vmn, jnp.minimum)
    hi_own = _xlane(vmx, jnp.maximum)

    mm[...] = vmn
    pltpu.sync_copy(mm, smin.at[pl.ds(pl.multiple_of(sid * L, 8), L)])
    mm[...] = vmx
    pltpu.sync_copy(mm, smax.at[pl.ds(pl.multiple_of(sid * L, 8), L)])
    plsc.subcore_barrier()

    pltpu.sync_copy(smin, mmall)
    accmin = mmall[pl.ds(0, L)]
    for l in range(1, NS):
        accmin = jnp.minimum(accmin, mmall[pl.ds(l * L, L)])
    core_lo = _xlane(accmin, jnp.minimum)
    pltpu.sync_copy(smax, mmall)
    accmax = mmall[pl.ds(0, L)]
    for l in range(1, NS):
        accmax = jnp.maximum(accmax, mmall[pl.ds(l * L, L)])
    core_hi = _xlane(accmax, jnp.maximum)

    lo = jnp.where(cid == 0, lo_own, core_lo)
    hi = jnp.where(cid == 0, hi_own, core_hi)
    scale = jnp.float32(NBINS) / (hi - lo)

    zvec = jnp.zeros((L,), jnp.float32)

    def zbody(k, _):
        hist[pl.ds(pl.multiple_of(k * L, 8), L)] = zvec
        return 0

    lax.fori_loop(0, PAD, zbody, 0)

    lane = lax.broadcasted_iota(jnp.int32, (L,), 0)
    ones = jnp.ones((L,), jnp.float32)

    def h_chunk(j, _):
        stage(j)

        def vbody(i, _):
            x = buf[pl.ds(pl.multiple_of(i * L, 8), L)]
            t = (x - lo) * scale
            b = t.astype(jnp.int32)
            b = jnp.minimum(b.astype(jnp.uint32),
                            jnp.uint32(NBINS - 1)).astype(jnp.int32)
            addr = (b << 4) | lane
            plsc.addupdate_scatter(hist, [addr], ones)
            return 0

        return lax.fori_loop(0, CH // L, vbody, 0, unroll=8)

    lax.fori_loop(0, NCH, h_chunk, 0)

    kidx = lax.broadcasted_iota(jnp.int32, (L,), 0) * L
    for g in range(PAD // L):
        acc = jnp.zeros((L,), jnp.float32)
        for l in range(L):
            acc = acc + plsc.load_gather(hist, [kidx + (g * L * L + l)])
        red[pl.ds(g * L, L)] = acc

    @pl.when(cid == 0)
    def _():
        pltpu.sync_copy(red, ph_hbm.at[sid])

    @pl.when(cid == 1)
    def _():
        pltpu.sync_copy(red, tp_hbm.at[sid])


_mesh = plsc.VectorSubcoreMesh(core_axis_name="c", subcore_axis_name="s",
                               num_cores=NC, num_subcores=NS)

_sc_hist = pl.kernel(
    _sc_body,
    out_type=(jax.ShapeDtypeStruct((ROWS, PAD), jnp.float32),
              jax.ShapeDtypeStruct((ROWS, PAD), jnp.float32)),
    mesh=_mesh,
    compiler_params=pltpu.CompilerParams(needs_layout_passes=False),
    scratch_types=[
        pltpu.VMEM((CH,), jnp.float32),
        pltpu.VMEM((PAD * L,), jnp.float32),
        pltpu.VMEM((PAD,), jnp.float32),
        pltpu.VMEM((L,), jnp.float32),
        pltpu.VMEM((NS * L,), jnp.float32),
        pltpu.VMEM_SHARED((NS * L,), jnp.float32),
        pltpu.VMEM_SHARED((NS * L,), jnp.float32),
    ],
)


def _loss_body(ph_ref, tp_ref, o_ref):
    ph = ph_ref[...]
    tp = tp_ref[...]
    cols = lax.broadcasted_iota(jnp.int32, (ROWS, PAD), 1)
    valid = cols < NBINS
    ph = jnp.where(valid, ph, 0.0)
    tp = jnp.where(valid, tp, 0.0)
    th = jnp.sum(tp, axis=0, keepdims=True)
    base = jnp.minimum(ph, th)
    safe = jnp.where(ph == 0.0, 1.0, ph)
    r = base / safe
    sim = jnp.sum(r * r, axis=1) / jnp.float32(NBINS)
    o_ref[0] = jnp.sum(1.0 - sim)


_loss_tc = pl.pallas_call(
    _loss_body,
    out_shape=jax.ShapeDtypeStruct((1,), jnp.float32),
    out_specs=pl.BlockSpec(memory_space=pltpu.SMEM),
)


def kernel(output, target):
    ph, tp = _sc_hist(output, target)
    loss = _loss_tc(ph, tp)
    return jnp.reshape(loss, ())


# speedup vs baseline: 125.0115x; 3.5549x over previous
import jax
import jax.numpy as jnp
from jax import lax
from jax.experimental import pallas as pl
from jax.experimental.pallas import tpu as pltpu
from jax.experimental.pallas import tpu_sc as plsc

NC = 2
NS = 16
L = 16
ROWS = 16
COLS = 1048576
NBINS = 100
PAD = 128
CH = 32768
NCH = COLS // CH


def _sc_body(out_hbm, tgt_hbm, ph_hbm, tp_hbm,
             buf0, buf1, hist, red, mm, mmall, smin, smax, sem0, sem1):
    cid = lax.axis_index("c")
    sid = lax.axis_index("s")
    bufs = (buf0, buf1)
    sems = (sem0, sem1)

    def start(j, b):
        off = pl.multiple_of(j * CH, 8)

        @pl.when(cid == 0)
        def _():
            pltpu.async_copy(out_hbm.at[sid, pl.ds(off, CH)], bufs[b], sems[b])

        @pl.when(cid == 1)
        def _():
            pltpu.async_copy(tgt_hbm.at[sid, pl.ds(off, CH)], bufs[b], sems[b])

    def wait(b):
        pltpu.make_async_copy(out_hbm.at[0, pl.ds(0, CH)], bufs[b],
                              sems[b]).wait()

    def two_pass(process, init_carry):
        def body2(k, carry):
            off = pl.multiple_of(k * CH, 8)

            @pl.when(cid == 0)
            def _():
                pltpu.sync_copy(out_hbm.at[sid, pl.ds(off, CH)], bufs[0])

            @pl.when(cid == 1)
            def _():
                pltpu.sync_copy(tgt_hbm.at[sid, pl.ds(off, CH)], bufs[0])

            return process(bufs[0], carry)

        return lax.fori_loop(0, NCH, body2, init_carry)

    def mm_process(buf, carry):
        def vbody(i, c):
            x = buf[pl.ds(pl.multiple_of(i * L, 8), L)]
            return jnp.minimum(c[0], x), jnp.maximum(c[1], x)

        return plsc.parallel_loop(0, CH // L, unroll=8, carry=carry)(vbody)

    init = (jnp.full((L,), jnp.inf, jnp.float32),
            jnp.full((L,), -jnp.inf, jnp.float32))
    vmn, vmx = two_pass(mm_process, init)

    gdn = lax.GatherDimensionNumbers(offset_dims=(), collapsed_slice_dims=(0,),
                                     start_index_map=(0,))

    def _xlane(v, op):
        for p in (8, 4, 2, 1):
            perm = (lax.broadcasted_iota(jnp.int32, (L,), 0) ^ p).reshape(L, 1)
            v = op(v, lax.gather(v, perm, gdn, slice_sizes=(1,),
                                 mode=lax.GatherScatterMode.PROMISE_IN_BOUNDS))
        return v

    lo_own = _xlane(vmn, jnp.minimum)
    hi_own = _xlane(vmx, jnp.maximum)

    mm[...] = vmn
    pltpu.sync_copy(mm, smin.at[pl.ds(pl.multiple_of(sid * L, 8), L)])
    mm[...] = vmx
    pltpu.sync_copy(mm, smax.at[pl.ds(pl.multiple_of(sid * L, 8), L)])
    plsc.subcore_barrier()

    pltpu.sync_copy(smin, mmall)
    accmin = mmall[pl.ds(0, L)]
    for l in range(1, NS):
        accmin = jnp.minimum(accmin, mmall[pl.ds(l * L, L)])
    core_lo = _xlane(accmin, jnp.minimum)
    pltpu.sync_copy(smax, mmall)
    accmax = mmall[pl.ds(0, L)]
    for l in range(1, NS):
        accmax = jnp.maximum(accmax, mmall[pl.ds(l * L, L)])
    core_hi = _xlane(accmax, jnp.maximum)

    lo = jnp.where(cid == 0, lo_own, core_lo)
    hi = jnp.where(cid == 0, hi_own, core_hi)
    scale = jnp.float32(NBINS) / (hi - lo)

    zvec = jnp.zeros((L,), jnp.float32)

    @plsc.parallel_loop(0, PAD, unroll=8)
    def _(k):
        hist[pl.ds(pl.multiple_of(k * L, 8), L)] = zvec

    lane = lax.broadcasted_iota(jnp.int32, (L,), 0)
    ones = jnp.ones((L,), jnp.float32)

    def h_process(buf, carry):
        @plsc.parallel_loop(0, CH // L, unroll=8)
        def _(i):
            x = buf[pl.ds(pl.multiple_of(i * L, 8), L)]
            t = (x - lo) * scale
            b = t.astype(jnp.int32)
            addr = (b << 4) | lane
            plsc.addupdate_scatter(hist, [addr], ones)

        return carry

    two_pass(h_process, 0)

    kidx = lax.broadcasted_iota(jnp.int32, (L,), 0) * L
    for g in range(PAD // L):
        acc = jnp.zeros((L,), jnp.float32)
        for l in range(L):
            acc = acc + plsc.load_gather(hist, [kidx + (g * L * L + l)])
        red[pl.ds(g * L, L)] = acc

    @pl.when(cid == 0)
    def _():
        pltpu.sync_copy(red, ph_hbm.at[sid])

    @pl.when(cid == 1)
    def _():
        pltpu.sync_copy(red, tp_hbm.at[sid])


_mesh = plsc.VectorSubcoreMesh(core_axis_name="c", subcore_axis_name="s",
                               num_cores=NC, num_subcores=NS)

_sc_hist = pl.kernel(
    _sc_body,
    out_type=(jax.ShapeDtypeStruct((ROWS, PAD), jnp.float32),
              jax.ShapeDtypeStruct((ROWS, PAD), jnp.float32)),
    mesh=_mesh,
    compiler_params=pltpu.CompilerParams(needs_layout_passes=False),
    scratch_types=[
        pltpu.VMEM((CH,), jnp.float32),
        pltpu.VMEM((CH,), jnp.float32),
        pltpu.VMEM((PAD * L,), jnp.float32),
        pltpu.VMEM((PAD,), jnp.float32),
        pltpu.VMEM((L,), jnp.float32),
        pltpu.VMEM((NS * L,), jnp.float32),
        pltpu.VMEM_SHARED((NS * L,), jnp.float32),
        pltpu.VMEM_SHARED((NS * L,), jnp.float32),
        pltpu.SemaphoreType.DMA,
        pltpu.SemaphoreType.DMA,
    ],
)


def _loss_body(ph_ref, tp_ref, o_ref):
    ph = ph_ref[...]
    tp = tp_ref[...]
    cols = lax.broadcasted_iota(jnp.int32, (ROWS, PAD), 1)

    def fold(h):
        over = jnp.sum(jnp.where(cols == NBINS, h, 0.0), axis=1, keepdims=True)
        h = jnp.where(cols == NBINS - 1, h + over, h)
        return jnp.where(cols < NBINS, h, 0.0)

    ph = fold(ph)
    tp = fold(tp)
    th = jnp.sum(tp, axis=0, keepdims=True)
    base = jnp.minimum(ph, th)
    safe = jnp.where(ph == 0.0, 1.0, ph)
    r = base / safe
    sim = jnp.sum(r * r, axis=1) / jnp.float32(NBINS)
    o_ref[0] = jnp.sum(1.0 - sim)


_loss_tc = pl.pallas_call(
    _loss_body,
    out_shape=jax.ShapeDtypeStruct((1,), jnp.float32),
    out_specs=pl.BlockSpec(memory_space=pltpu.SMEM),
)


def kernel(output, target):
    ph, tp = _sc_hist(output, target)
    loss = _loss_tc(ph, tp)
    return jnp.reshape(loss, ())


# speedup vs baseline: 189.6151x; 1.5168x over previous
import jax
import jax.numpy as jnp
from jax import lax
from jax.experimental import pallas as pl
from jax.experimental.pallas import tpu as pltpu
from jax.experimental.pallas import tpu_sc as plsc

NC = 2
NS = 16
L = 16
ROWS = 16
COLS = 1048576
NBINS = 100
PAD = 128
CH = 32768
NCH = COLS // CH


def _sc_body(out_hbm, tgt_hbm, ph_hbm, tp_hbm,
             buf0, buf1, hist, red, mm, mm2, mmall, smin, smax, sem0, sem1):
    cid = lax.axis_index("c")
    sid = lax.axis_index("s")
    bufs = (buf0, buf1)
    sems = (sem0, sem1)
    carry_refs = (mm, mm2)

    def _pipe(src_hbm, process, carry):
        def start(j, b):
            off = pl.multiple_of(j * CH, 8)
            return pltpu.async_copy(src_hbm.at[sid, pl.ds(off, CH)],
                                    bufs[b], sems[b])

        descs = [start(0, 0), None]
        for j in range(NCH):
            b = j % 2
            if j + 1 < NCH:
                descs[1 - b] = start(j + 1, 1 - b)
            descs[b].wait()
            carry = process(bufs[b], carry)
        return carry

    def two_pass(process, init_carry):
        @pl.when(cid == 0)
        def _():
            res = _pipe(out_hbm, process, init_carry)
            for r, v in zip(carry_refs, jax.tree.leaves(res)):
                r[...] = v

        @pl.when(cid == 1)
        def _():
            res = _pipe(tgt_hbm, process, init_carry)
            for r, v in zip(carry_refs, jax.tree.leaves(res)):
                r[...] = v

        return jax.tree.unflatten(
            jax.tree.structure(init_carry),
            [r[...] for r in carry_refs[:len(jax.tree.leaves(init_carry))]])

    def mm_process(buf, carry):
        def vbody(i, c):
            x = buf[pl.ds(pl.multiple_of(i * L, 8), L)]
            return jnp.minimum(c[0], x), jnp.maximum(c[1], x)

        return plsc.parallel_loop(0, CH // L, unroll=8, carry=carry)(vbody)

    init = (jnp.full((L,), jnp.inf, jnp.float32),
            jnp.full((L,), -jnp.inf, jnp.float32))
    vmn, vmx = two_pass(mm_process, init)

    gdn = lax.GatherDimensionNumbers(offset_dims=(), collapsed_slice_dims=(0,),
                                     start_index_map=(0,))

    def _xlane(v, op):
        for p in (8, 4, 2, 1):
            perm = (lax.broadcasted_iota(jnp.int32, (L,), 0) ^ p).reshape(L, 1)
            v = op(v, lax.gather(v, perm, gdn, slice_sizes=(1,),
                                 mode=lax.GatherScatterMode.PROMISE_IN_BOUNDS))
        return v

    lo_own = _xlane(vmn, jnp.minimum)
    hi_own = _xlane(vmx, jnp.maximum)

    mm[...] = vmn
    pltpu.sync_copy(mm, smin.at[pl.ds(pl.multiple_of(sid * L, 8), L)])
    mm[...] = vmx
    pltpu.sync_copy(mm, smax.at[pl.ds(pl.multiple_of(sid * L, 8), L)])
    plsc.subcore_barrier()

    pltpu.sync_copy(smin, mmall)
    accmin = mmall[pl.ds(0, L)]
    for l in range(1, NS):
        accmin = jnp.minimum(accmin, mmall[pl.ds(l * L, L)])
    core_lo = _xlane(accmin, jnp.minimum)
    pltpu.sync_copy(smax, mmall)
    accmax = mmall[pl.ds(0, L)]
    for l in range(1, NS):
        accmax = jnp.maximum(accmax, mmall[pl.ds(l * L, L)])
    core_hi = _xlane(accmax, jnp.maximum)

    lo = jnp.where(cid == 0, lo_own, core_lo)
    hi = jnp.where(cid == 0, hi_own, core_hi)
    scale = jnp.float32(NBINS) / (hi - lo)

    zvec = jnp.zeros((L,), jnp.float32)

    @plsc.parallel_loop(0, PAD, unroll=8)
    def _(k):
        hist[pl.ds(pl.multiple_of(k * L, 8), L)] = zvec

    lane = lax.broadcasted_iota(jnp.int32, (L,), 0)
    ones = jnp.ones((L,), jnp.float32)

    def h_process(buf, carry):
        @plsc.parallel_loop(0, CH // L, unroll=8)
        def _(i):
            x = buf[pl.ds(pl.multiple_of(i * L, 8), L)]
            t = (x - lo) * scale
            b = t.astype(jnp.int32)
            addr = (b << 4) | lane
            plsc.addupdate_scatter(hist, [addr], ones)

        return carry

    two_pass(h_process, ())

    kidx = lax.broadcasted_iota(jnp.int32, (L,), 0) * L
    for g in range(PAD // L):
        acc = jnp.zeros((L,), jnp.float32)
        for l in range(L):
            acc = acc + plsc.load_gather(hist, [kidx + (g * L * L + l)])
        red[pl.ds(g * L, L)] = acc

    @pl.when(cid == 0)
    def _():
        pltpu.sync_copy(red, ph_hbm.at[sid])

    @pl.when(cid == 1)
    def _():
        pltpu.sync_copy(red, tp_hbm.at[sid])


_mesh = plsc.VectorSubcoreMesh(core_axis_name="c", subcore_axis_name="s",
                               num_cores=NC, num_subcores=NS)

_sc_hist = pl.kernel(
    _sc_body,
    out_type=(jax.ShapeDtypeStruct((ROWS, PAD), jnp.float32),
              jax.ShapeDtypeStruct((ROWS, PAD), jnp.float32)),
    mesh=_mesh,
    compiler_params=pltpu.CompilerParams(needs_layout_passes=False),
    scratch_types=[
        pltpu.VMEM((CH,), jnp.float32),
        pltpu.VMEM((CH,), jnp.float32),
        pltpu.VMEM((PAD * L,), jnp.float32),
        pltpu.VMEM((PAD,), jnp.float32),
        pltpu.VMEM((L,), jnp.float32),
        pltpu.VMEM((L,), jnp.float32),
        pltpu.VMEM((NS * L,), jnp.float32),
        pltpu.VMEM_SHARED((NS * L,), jnp.float32),
        pltpu.VMEM_SHARED((NS * L,), jnp.float32),
        pltpu.SemaphoreType.DMA,
        pltpu.SemaphoreType.DMA,
    ],
)


def _loss_body(ph_ref, tp_ref, o_ref):
    ph = ph_ref[...]
    tp = tp_ref[...]
    cols = lax.broadcasted_iota(jnp.int32, (ROWS, PAD), 1)

    def fold(h):
        over = jnp.sum(jnp.where(cols == NBINS, h, 0.0), axis=1, keepdims=True)
        h = jnp.where(cols == NBINS - 1, h + over, h)
        return jnp.where(cols < NBINS, h, 0.0)

    ph = fold(ph)
    tp = fold(tp)
    th = jnp.sum(tp, axis=0, keepdims=True)
    base = jnp.minimum(ph, th)
    safe = jnp.where(ph == 0.0, 1.0, ph)
    r = base / safe
    sim = jnp.sum(r * r, axis=1) / jnp.float32(NBINS)
    o_ref[0] = jnp.sum(1.0 - sim)


_loss_tc = pl.pallas_call(
    _loss_body,
    out_shape=jax.ShapeDtypeStruct((1,), jnp.float32),
    out_specs=pl.BlockSpec(memory_space=pltpu.SMEM),
)


def kernel(output, target):
    ph, tp = _sc_hist(output, target)
    loss = _loss_tc(ph, tp)
    return jnp.reshape(loss, ())


# speedup vs baseline: 199.2622x; 1.0509x over previous
import jax
import jax.numpy as jnp
from jax import lax
from jax.experimental import pallas as pl
from jax.experimental.pallas import tpu as pltpu
from jax.experimental.pallas import tpu_sc as plsc

NC = 2
NS = 16
L = 16
ROWS = 16
COLS = 1048576
NBINS = 100
PAD = 128
CH = 32768
NCH = COLS // CH


def _sc_body(out_hbm, tgt_hbm, ph_hbm, tp_hbm,
             buf0, buf1, buf2, hist, red, mm, mm2, mmall, smin, smax,
             sem0, sem1, sem2):
    cid = lax.axis_index("c")
    sid = lax.axis_index("s")
    bufs = (buf0, buf1, buf2)
    sems = (sem0, sem1, sem2)
    carry_refs = (mm, mm2)

    def _pipe(src_hbm, process, carry):
        def start(j, b):
            off = pl.multiple_of(j * CH, 8)
            return pltpu.async_copy(src_hbm.at[sid, pl.ds(off, CH)],
                                    bufs[b], sems[b])

        nbuf = len(bufs)
        descs = [start(j, j) for j in range(nbuf)]
        for j in range(NCH):
            b = j % nbuf
            descs[b].wait()
            carry = process(bufs[b], carry)
            if j + nbuf < NCH:
                descs[b] = start(j + nbuf, b)
        return carry

    def two_pass(process, init_carry):
        @pl.when(cid == 0)
        def _():
            res = _pipe(out_hbm, process, init_carry)
            for r, v in zip(carry_refs, jax.tree.leaves(res)):
                r[...] = v

        @pl.when(cid == 1)
        def _():
            res = _pipe(tgt_hbm, process, init_carry)
            for r, v in zip(carry_refs, jax.tree.leaves(res)):
                r[...] = v

        return jax.tree.unflatten(
            jax.tree.structure(init_carry),
            [r[...] for r in carry_refs[:len(jax.tree.leaves(init_carry))]])

    def mm_process(buf, carry):
        def vbody(i, c):
            x = buf[pl.ds(pl.multiple_of(i * L, 8), L)]
            return jnp.minimum(c[0], x), jnp.maximum(c[1], x)

        return plsc.parallel_loop(0, CH // L, unroll=8, carry=carry)(vbody)

    init = (jnp.full((L,), jnp.inf, jnp.float32),
            jnp.full((L,), -jnp.inf, jnp.float32))
    vmn, vmx = two_pass(mm_process, init)

    gdn = lax.GatherDimensionNumbers(offset_dims=(), collapsed_slice_dims=(0,),
                                     start_index_map=(0,))

    def _xlane(v, op):
        for p in (8, 4, 2, 1):
            perm = (lax.broadcasted_iota(jnp.int32, (L,), 0) ^ p).reshape(L, 1)
            v = op(v, lax.gather(v, perm, gdn, slice_sizes=(1,),
                                 mode=lax.GatherScatterMode.PROMISE_IN_BOUNDS))
        return v

    lo_own = _xlane(vmn, jnp.minimum)
    hi_own = _xlane(vmx, jnp.maximum)

    mm[...] = vmn
    pltpu.sync_copy(mm, smin.at[pl.ds(pl.multiple_of(sid * L, 8), L)])
    mm[...] = vmx
    pltpu.sync_copy(mm, smax.at[pl.ds(pl.multiple_of(sid * L, 8), L)])
    plsc.subcore_barrier()

    pltpu.sync_copy(smin, mmall)
    accmin = mmall[pl.ds(0, L)]
    for l in range(1, NS):
        accmin = jnp.minimum(accmin, mmall[pl.ds(l * L, L)])
    core_lo = _xlane(accmin, jnp.minimum)
    pltpu.sync_copy(smax, mmall)
    accmax = mmall[pl.ds(0, L)]
    for l in range(1, NS):
        accmax = jnp.maximum(accmax, mmall[pl.ds(l * L, L)])
    core_hi = _xlane(accmax, jnp.maximum)

    lo = jnp.where(cid == 0, lo_own, core_lo)
    hi = jnp.where(cid == 0, hi_own, core_hi)
    scale = jnp.float32(NBINS) / (hi - lo)

    zvec = jnp.zeros((L,), jnp.float32)

    @plsc.parallel_loop(0, PAD, unroll=8)
    def _(k):
        hist[pl.ds(pl.multiple_of(k * L, 8), L)] = zvec

    lane = lax.broadcasted_iota(jnp.int32, (L,), 0)
    ones = jnp.ones((L,), jnp.float32)

    def h_process(buf, carry):
        @plsc.parallel_loop(0, CH // L, unroll=8)
        def _(i):
            x = buf[pl.ds(pl.multiple_of(i * L, 8), L)]
            t = (x - lo) * scale
            b = t.astype(jnp.int32)
            addr = (b << 4) | lane
            plsc.addupdate_scatter(hist, [addr], ones)

        return carry

    two_pass(h_process, ())

    kidx = lax.broadcasted_iota(jnp.int32, (L,), 0) * L
    for g in range(PAD // L):
        acc = jnp.zeros((L,), jnp.float32)
        for l in range(L):
            acc = acc + plsc.load_gather(hist, [kidx + (g * L * L + l)])
        red[pl.ds(g * L, L)] = acc

    @pl.when(cid == 0)
    def _():
        pltpu.sync_copy(red, ph_hbm.at[sid])

    @pl.when(cid == 1)
    def _():
        pltpu.sync_copy(red, tp_hbm.at[sid])


_mesh = plsc.VectorSubcoreMesh(core_axis_name="c", subcore_axis_name="s",
                               num_cores=NC, num_subcores=NS)

_sc_hist = pl.kernel(
    _sc_body,
    out_type=(jax.ShapeDtypeStruct((ROWS, PAD), jnp.float32),
              jax.ShapeDtypeStruct((ROWS, PAD), jnp.float32)),
    mesh=_mesh,
    compiler_params=pltpu.CompilerParams(needs_layout_passes=False),
    scratch_types=[
        pltpu.VMEM((CH,), jnp.float32),
        pltpu.VMEM((CH,), jnp.float32),
        pltpu.VMEM((CH,), jnp.float32),
        pltpu.VMEM((PAD * L,), jnp.float32),
        pltpu.VMEM((PAD,), jnp.float32),
        pltpu.VMEM((L,), jnp.float32),
        pltpu.VMEM((L,), jnp.float32),
        pltpu.VMEM((NS * L,), jnp.float32),
        pltpu.VMEM_SHARED((NS * L,), jnp.float32),
        pltpu.VMEM_SHARED((NS * L,), jnp.float32),
        pltpu.SemaphoreType.DMA,
        pltpu.SemaphoreType.DMA,
        pltpu.SemaphoreType.DMA,
    ],
)


def _loss_body(ph_ref, tp_ref, o_ref):
    ph = ph_ref[...]
    tp = tp_ref[...]
    cols = lax.broadcasted_iota(jnp.int32, (ROWS, PAD), 1)

    def fold(h):
        over = jnp.sum(jnp.where(cols == NBINS, h, 0.0), axis=1, keepdims=True)
        h = jnp.where(cols == NBINS - 1, h + over, h)
        return jnp.where(cols < NBINS, h, 0.0)

    ph = fold(ph)
    tp = fold(tp)
    th = jnp.sum(tp, axis=0, keepdims=True)
    base = jnp.minimum(ph, th)
    safe = jnp.where(ph == 0.0, 1.0, ph)
    r = base / safe
    sim = jnp.sum(r * r, axis=1) / jnp.float32(NBINS)
    o_ref[0] = jnp.sum(1.0 - sim)


_loss_tc = pl.pallas_call(
    _loss_body,
    out_shape=jax.ShapeDtypeStruct((1,), jnp.float32),
    out_specs=pl.BlockSpec(memory_space=pltpu.SMEM),
)


def kernel(output, target):
    ph, tp = _sc_hist(output, target)
    loss = _loss_tc(ph, tp)
    return jnp.reshape(loss, ())
